# baseline (speedup 1.0000x reference)
import jax
import jax.numpy as jnp
from jax.experimental import pallas as pl
from jax.experimental.pallas import tpu as pltpu

_ROWS = 100000
_D = 128

_SIZES = [1000, 2000, 4000] + [5000] * 18 + [2000, 1000]
assert sum(_SIZES) == _ROWS and all(s % 8 == 0 for s in _SIZES)
_N = len(_SIZES)
_BASES = [sum(_SIZES[:i]) for i in range(_N)]


def _body(xs_ref, x_ref, out_ref, *scratch):
    bufs = scratch[:_N]
    rsems = scratch[_N]
    wsems = scratch[_N + 1]

    reads = []
    for i in range(_N):
        cp = pltpu.make_async_copy(
            xs_ref.at[pl.ds(_BASES[i], _SIZES[i])], bufs[i], rsems.at[i])
        cp.start()
        reads.append(cp)

    writes = []
    for i in range(_N):
        reads[i].wait()
        if i == 0:
            bufs[0][0:1, :] = x_ref[...]
        cp = pltpu.make_async_copy(
            bufs[i], out_ref.at[pl.ds(_BASES[i], _SIZES[i])], wsems.at[i])
        cp.start()
        writes.append(cp)
    for cp in writes:
        cp.wait()


@jax.jit
def kernel(xs, x):
    return pl.pallas_call(
        _body,
        out_shape=jax.ShapeDtypeStruct((_ROWS, _D), jnp.float32),
        in_specs=[
            pl.BlockSpec(memory_space=pl.ANY),
            pl.BlockSpec(memory_space=pltpu.VMEM),
        ],
        out_specs=pl.BlockSpec(memory_space=pl.ANY),
        scratch_shapes=(
            [pltpu.VMEM((s, _D), jnp.float32) for s in _SIZES]
            + [pltpu.SemaphoreType.DMA((_N,)), pltpu.SemaphoreType.DMA((_N,))]
        ),
    )(xs, x)


# speedup vs baseline: 1.0315x; 1.0315x over previous
import jax
import jax.numpy as jnp
from jax.experimental import pallas as pl
from jax.experimental.pallas import tpu as pltpu

_ROWS = 100000
_D = 128

_SIZES = [2000, 6000, 20000, 20000, 20000, 20000, 8000, 4000]
assert sum(_SIZES) == _ROWS and all(s % 8 == 0 for s in _SIZES)
_N = len(_SIZES)
_BASES = [sum(_SIZES[:i]) for i in range(_N)]


def _body(xs_ref, x_ref, out_ref, *scratch):
    bufs = scratch[:_N]
    rsems = scratch[_N]
    wsems = scratch[_N + 1]

    reads = []
    for i in range(_N):
        cp = pltpu.make_async_copy(
            xs_ref.at[pl.ds(_BASES[i], _SIZES[i])], bufs[i], rsems.at[i])
        cp.start()
        reads.append(cp)

    writes = []
    for i in range(_N):
        reads[i].wait()
        if i == 0:
            bufs[0][0:1, :] = x_ref[...]
        cp = pltpu.make_async_copy(
            bufs[i], out_ref.at[pl.ds(_BASES[i], _SIZES[i])], wsems.at[i])
        cp.start()
        writes.append(cp)
    for cp in writes:
        cp.wait()


@jax.jit
def kernel(xs, x):
    return pl.pallas_call(
        _body,
        out_shape=jax.ShapeDtypeStruct((_ROWS, _D), jnp.float32),
        in_specs=[
            pl.BlockSpec(memory_space=pl.ANY),
            pl.BlockSpec(memory_space=pltpu.VMEM),
        ],
        out_specs=pl.BlockSpec(memory_space=pl.ANY),
        scratch_shapes=(
            [pltpu.VMEM((s, _D), jnp.float32) for s in _SIZES]
            + [pltpu.SemaphoreType.DMA((_N,)), pltpu.SemaphoreType.DMA((_N,))]
        ),
    )(xs, x)


# speedup vs baseline: 1.0393x; 1.0076x over previous
import jax
import jax.numpy as jnp
from jax.experimental import pallas as pl
from jax.experimental.pallas import tpu as pltpu

_ROWS = 100000
_D = 128
_BS = 20000
_GRID = _ROWS // _BS


def _body(xs_ref, x_ref, out_ref):
    out_ref[...] = xs_ref[...]

    @pl.when(pl.program_id(0) == 0)
    def _():
        out_ref[0:1, :] = x_ref[...]


@jax.jit
def kernel(xs, x):
    return pl.pallas_call(
        _body,
        grid=(_GRID,),
        out_shape=jax.ShapeDtypeStruct((_ROWS, _D), jnp.float32),
        in_specs=[
            pl.BlockSpec((_BS, _D), lambda i: (i, 0)),
            pl.BlockSpec((1, _D), lambda i: (0, 0)),
        ],
        out_specs=pl.BlockSpec((_BS, _D), lambda i: (i, 0)),
        compiler_params=pltpu.CompilerParams(
            dimension_semantics=("arbitrary",),
        ),
    )(xs, x)


# speedup vs baseline: 1.0404x; 1.0011x over previous
import jax
import jax.numpy as jnp
from jax.experimental import pallas as pl
from jax.experimental.pallas import tpu as pltpu

_ROWS = 100000
_D = 128
_BS = 20000
_GRID = _ROWS // _BS


def _body(xs_ref, x_ref, out_ref):
    out_ref[...] = xs_ref[...]

    @pl.when(pl.program_id(0) == 0)
    def _():
        out_ref[0:1, :] = x_ref[...]


@jax.jit
def kernel(xs, x):
    return pl.pallas_call(
        _body,
        grid=(_GRID,),
        out_shape=jax.ShapeDtypeStruct((_ROWS, _D), jnp.float32),
        in_specs=[
            pl.BlockSpec((_BS, _D), lambda i: (i, 0)),
            pl.BlockSpec((1, _D), lambda i: (0, 0)),
        ],
        out_specs=pl.BlockSpec((_BS, _D), lambda i: (i, 0)),
        compiler_params=pltpu.CompilerParams(
            dimension_semantics=("parallel",),
        ),
    )(xs, x)
